# baseline (speedup 1.0000x reference)
import jax
import jax.numpy as jnp
from jax import lax
from jax.experimental import pallas as pl
from jax.experimental.pallas import tpu as pltpu
from jax.experimental.pallas import tpu_sc as plsc

NC, NS = 2, 16
NW = NC * NS
B, L, DIM = 16384, 50, 32
NUM_ROWS = 1000000
BAGS_W = B // NW
CB = 32
NCH = BAGS_W // CB
NST = NCH // 2
RPC = CB * L
INV_L = 1.0 / L


def _body(tok_hbm, tab_hbm, out_hbm,
          idx0, idx1, rows0, rows1, out0, out1, sem0, sem1):
    wid = lax.axis_index("s") * NC + lax.axis_index("c")
    idx = (idx0, idx1)
    rows = (rows0, rows1)
    outs = (out0, out1)
    sems = (sem0, sem1)

    def fire(c, p):
        bag0 = wid * BAGS_W + c * CB
        pltpu.sync_copy(tok_hbm.at[pl.ds(bag0 * L, RPC)], idx[p])
        pltpu.async_copy(tab_hbm.at[idx[p]], rows[p], sems[p])

    def process(c, p):
        pltpu.make_async_copy(tab_hbm.at[idx[p]], rows[p], sems[p]).wait()
        rv = rows[p]
        ov = outs[p]

        def bag(b, carry):
            a0 = rv[b * L, 0:16]
            a1 = rv[b * L, 16:32]
            for r in range(1, L):
                a0 = a0 + rv[b * L + r, 0:16]
                a1 = a1 + rv[b * L + r, 16:32]
            ov[b, 0:16] = a0 * INV_L
            ov[b, 16:32] = a1 * INV_L
            return carry

        lax.fori_loop(0, CB, bag, 0)
        bag0 = wid * BAGS_W + c * CB
        pltpu.sync_copy(ov, out_hbm.at[pl.ds(bag0, CB)])

    fire(0, 0)

    def step(s, carry):
        c0 = s * 2
        fire(c0 + 1, 1)
        process(c0, 0)

        @pl.when(s < NST - 1)
        def _():
            fire(c0 + 2, 0)

        process(c0 + 1, 1)
        return carry

    lax.fori_loop(0, NST, step, 0)


def kernel(tokens_idx, emb_weight):
    tok = tokens_idx.reshape(-1).astype(jnp.int32)
    tab = lax.optimization_barrier(emb_weight.reshape(-1)).reshape(
        NUM_ROWS, DIM)
    mesh = plsc.VectorSubcoreMesh(core_axis_name="c", subcore_axis_name="s")
    f = pl.kernel(
        _body,
        out_type=jax.ShapeDtypeStruct((B, DIM), jnp.float32),
        mesh=mesh,
        compiler_params=pltpu.CompilerParams(use_tc_tiling_on_sc=False),
        scratch_types=[
            pltpu.VMEM((RPC,), jnp.int32),
            pltpu.VMEM((RPC,), jnp.int32),
            pltpu.VMEM((RPC, DIM), jnp.float32),
            pltpu.VMEM((RPC, DIM), jnp.float32),
            pltpu.VMEM((CB, DIM), jnp.float32),
            pltpu.VMEM((CB, DIM), jnp.float32),
            pltpu.SemaphoreType.DMA,
            pltpu.SemaphoreType.DMA,
        ],
    )
    return f(tok, tab)


# speedup vs baseline: 1.6387x; 1.6387x over previous
import jax
import jax.numpy as jnp
from jax import lax
from jax.experimental import pallas as pl
from jax.experimental.pallas import tpu as pltpu
from jax.experimental.pallas import tpu_sc as plsc

NC, NS = 2, 16
NW = NC * NS
B, L, DIM = 16384, 50, 32
NUM_ROWS = 1000000
BAGS_W = B // NW
CB = 32
NCH = BAGS_W // CB
NST = NCH // 2
RPC = CB * L
INV_L = 1.0 / L


def _body(tok_hbm, tab_hbm, out_hbm,
          idx0, idx1, rows0, rows1, out0, out1, sem0, sem1):
    wid = lax.axis_index("s") * NC + lax.axis_index("c")
    idx = (idx0, idx1)
    rows = (rows0, rows1)
    outs = (out0, out1)
    sems = (sem0, sem1)

    def fire(c, p):
        bag0 = wid * BAGS_W + c * CB
        pltpu.sync_copy(tok_hbm.at[pl.ds(bag0 * L, RPC)], idx[p])
        pltpu.async_copy(tab_hbm.at[idx[p]], rows[p], sems[p])

    def process(c, p):
        pltpu.make_async_copy(tab_hbm.at[idx[p]], rows[p], sems[p]).wait()
        rv = rows[p]
        ov = outs[p]

        def bag(b, carry):
            a0 = rv[b * L, 0:16]
            a1 = rv[b * L, 16:32]
            for r in range(1, L):
                a0 = a0 + rv[b * L + r, 0:16]
                a1 = a1 + rv[b * L + r, 16:32]
            ov[b, 0:16] = a0 * INV_L
            ov[b, 16:32] = a1 * INV_L
            return carry

        lax.fori_loop(0, CB, bag, 0)
        bag0 = wid * BAGS_W + c * CB
        pltpu.sync_copy(ov, out_hbm.at[pl.ds(bag0, CB)])

    fire(0, 0)

    def step(s, carry):
        c0 = s * 2
        fire(c0 + 1, 1)
        process(c0, 0)

        @pl.when(s < NST - 1)
        def _():
            fire(c0 + 2, 0)

        process(c0 + 1, 1)
        return carry

    lax.fori_loop(0, NST, step, 0)


TBLK = 8192
TGRID = -(-NUM_ROWS // TBLK)
TROWS = TBLK * DIM // 128


def _transpose_body(x_ref, o_ref):
    for j in range(4):
        o_ref[:, 32 * j:32 * (j + 1)] = jnp.transpose(
            x_ref[:, TROWS * j:TROWS * (j + 1)])


def _linearize_table(emb_weight):
    tab_t = emb_weight.T
    out = pl.pallas_call(
        _transpose_body,
        grid=(TGRID,),
        in_specs=[pl.BlockSpec((DIM, TBLK), lambda g: (0, g))],
        out_specs=pl.BlockSpec((TROWS, 128), lambda g: (g, 0)),
        out_shape=jax.ShapeDtypeStruct((TGRID * TROWS, 128), jnp.float32),
    )(tab_t)
    return out.reshape(TGRID * TBLK, DIM)


def _permute_tokens(tok):
    g = tok // TBLK
    w = tok % TBLK
    return (g * TROWS + w % TROWS) * 4 + w // TROWS


def kernel(tokens_idx, emb_weight):
    tok = _permute_tokens(tokens_idx.reshape(-1).astype(jnp.int32))
    tab = _linearize_table(emb_weight)
    mesh = plsc.VectorSubcoreMesh(core_axis_name="c", subcore_axis_name="s")
    f = pl.kernel(
        _body,
        out_type=jax.ShapeDtypeStruct((B, DIM), jnp.float32),
        mesh=mesh,
        compiler_params=pltpu.CompilerParams(use_tc_tiling_on_sc=False),
        scratch_types=[
            pltpu.VMEM((RPC,), jnp.int32),
            pltpu.VMEM((RPC,), jnp.int32),
            pltpu.VMEM((RPC, DIM), jnp.float32),
            pltpu.VMEM((RPC, DIM), jnp.float32),
            pltpu.VMEM((CB, DIM), jnp.float32),
            pltpu.VMEM((CB, DIM), jnp.float32),
            pltpu.SemaphoreType.DMA,
            pltpu.SemaphoreType.DMA,
        ],
    )
    return f(tok, tab)


# speedup vs baseline: 2.4031x; 1.4664x over previous
import jax
import jax.numpy as jnp
from jax import lax
from jax.experimental import pallas as pl
from jax.experimental.pallas import tpu as pltpu
from jax.experimental.pallas import tpu_sc as plsc

NC, NS = 2, 16
NW = NC * NS
B, L, DIM = 16384, 50, 32
NUM_ROWS = 1000000
BAGS_W = B // NW
CB = 32
NCH = BAGS_W // CB
NST = NCH // 2
RPC = CB * L
INV_L = 1.0 / L


def _body(tok_hbm, tab_hbm, out_hbm,
          idx0, idx1, rows0, rows1, out0, out1, sem0, sem1):
    wid = lax.axis_index("s") * NC + lax.axis_index("c")
    idx = (idx0, idx1)
    rows = (rows0, rows1)
    outs = (out0, out1)
    sems = (sem0, sem1)

    def fire(c, p):
        bag0 = wid * BAGS_W + c * CB
        pltpu.sync_copy(tok_hbm.at[pl.ds(bag0 * L, RPC)], idx[p])
        pltpu.async_copy(tab_hbm.at[idx[p]], rows[p], sems[p])

    def process(c, p):
        pltpu.make_async_copy(tab_hbm.at[idx[p]], rows[p], sems[p]).wait()
        rv = rows[p]
        ov = outs[p]

        def bag(b, carry):
            a0 = rv[b * L, 0:16]
            a1 = rv[b * L, 16:32]
            for r in range(1, L):
                a0 = a0 + rv[b * L + r, 0:16]
                a1 = a1 + rv[b * L + r, 16:32]
            ov[b, 0:16] = a0 * INV_L
            ov[b, 16:32] = a1 * INV_L
            return carry

        lax.fori_loop(0, CB, bag, 0)
        bag0 = wid * BAGS_W + c * CB
        pltpu.sync_copy(ov, out_hbm.at[pl.ds(bag0, CB)])

    fire(0, 0)

    def step(s, carry):
        c0 = s * 2
        fire(c0 + 1, 1)
        process(c0, 0)

        @pl.when(s < NST - 1)
        def _():
            fire(c0 + 2, 0)

        process(c0 + 1, 1)
        return carry

    lax.fori_loop(0, NST, step, 0)


TBLK = 8192
TGRID = -(-NUM_ROWS // TBLK)
TROWS = TBLK * DIM // 128


def _transpose_body(x_ref, o_ref):
    x = x_ref[...]
    xx = jnp.concatenate(
        [x[:, TROWS * j:TROWS * (j + 1)] for j in range(4)], axis=0)
    o_ref[...] = jnp.transpose(xx)


def _linearize_table(emb_weight):
    tab_t = emb_weight.T
    out = pl.pallas_call(
        _transpose_body,
        grid=(TGRID,),
        in_specs=[pl.BlockSpec((DIM, TBLK), lambda g: (0, g))],
        out_specs=pl.BlockSpec((TROWS, 128), lambda g: (g, 0)),
        out_shape=jax.ShapeDtypeStruct((TGRID * TROWS, 128), jnp.float32),
    )(tab_t)
    return out.reshape(TGRID * TBLK, DIM)


def _permute_tokens(tok):
    g = tok // TBLK
    w = tok % TBLK
    return (g * TROWS + w % TROWS) * 4 + w // TROWS


def kernel(tokens_idx, emb_weight):
    tok = _permute_tokens(tokens_idx.reshape(-1).astype(jnp.int32))
    tab = _linearize_table(emb_weight)
    mesh = plsc.VectorSubcoreMesh(core_axis_name="c", subcore_axis_name="s")
    f = pl.kernel(
        _body,
        out_type=jax.ShapeDtypeStruct((B, DIM), jnp.float32),
        mesh=mesh,
        compiler_params=pltpu.CompilerParams(use_tc_tiling_on_sc=False),
        scratch_types=[
            pltpu.VMEM((RPC,), jnp.int32),
            pltpu.VMEM((RPC,), jnp.int32),
            pltpu.VMEM((RPC, DIM), jnp.float32),
            pltpu.VMEM((RPC, DIM), jnp.float32),
            pltpu.VMEM((CB, DIM), jnp.float32),
            pltpu.VMEM((CB, DIM), jnp.float32),
            pltpu.SemaphoreType.DMA,
            pltpu.SemaphoreType.DMA,
        ],
    )
    return f(tok, tab)


# speedup vs baseline: 2.8416x; 1.1825x over previous
import jax
import jax.numpy as jnp
from jax import lax
from jax.experimental import pallas as pl
from jax.experimental.pallas import tpu as pltpu
from jax.experimental.pallas import tpu_sc as plsc

NC, NS = 2, 16
NW = NC * NS
B, L, DIM = 16384, 50, 32
NUM_ROWS = 1000000
BAGS_W = B // NW
CB = 32
NCH = BAGS_W // CB
NST = NCH // 2
RPC = CB * L
INV_L = 1.0 / L


def _body(tok_hbm, tab_hbm, out_hbm,
          idx0, idx1, rows0, rows1, out0, out1, sem0, sem1):
    wid = lax.axis_index("s") * NC + lax.axis_index("c")
    idx = (idx0, idx1)
    rows = (rows0, rows1)
    outs = (out0, out1)
    sems = (sem0, sem1)

    def fire(c, p):
        bag0 = wid * BAGS_W + c * CB
        pltpu.sync_copy(tok_hbm.at[pl.ds(bag0 * L, RPC)], idx[p])
        pltpu.async_copy(tab_hbm.at[idx[p]], rows[p], sems[p])

    def process(c, p):
        pltpu.make_async_copy(tab_hbm.at[idx[p]], rows[p], sems[p]).wait()
        rv = rows[p]
        ov = outs[p]

        def bag(b, carry):
            a0 = rv[b * L, 0:16]
            a1 = rv[b * L, 16:32]
            for r in range(1, L):
                a0 = a0 + rv[b * L + r, 0:16]
                a1 = a1 + rv[b * L + r, 16:32]
            ov[b, 0:16] = a0 * INV_L
            ov[b, 16:32] = a1 * INV_L
            return carry

        lax.fori_loop(0, CB, bag, 0)
        bag0 = wid * BAGS_W + c * CB
        pltpu.sync_copy(ov, out_hbm.at[pl.ds(bag0, CB)])

    fire(0, 0)

    def step(s, carry):
        c0 = s * 2
        fire(c0 + 1, 1)
        process(c0, 0)

        @pl.when(s < NST - 1)
        def _():
            fire(c0 + 2, 0)

        process(c0 + 1, 1)
        return carry

    lax.fori_loop(0, NST, step, 0)


TBLK = 16384
TGRID = -(-NUM_ROWS // TBLK)
TROWS = TBLK * DIM // 128


def _transpose_body(x_ref, o_ref):
    x = x_ref[...]
    xx = jnp.concatenate(
        [x[:, TROWS * j:TROWS * (j + 1)] for j in range(4)], axis=0)
    o_ref[...] = jnp.transpose(xx)


def _linearize_table(emb_weight):
    tab_t = emb_weight.T
    out = pl.pallas_call(
        _transpose_body,
        grid=(TGRID,),
        in_specs=[pl.BlockSpec((DIM, TBLK), lambda g: (0, g))],
        out_specs=pl.BlockSpec((TROWS, 128), lambda g: (g, 0)),
        out_shape=jax.ShapeDtypeStruct((TGRID * TROWS, 128), jnp.float32),
    )(tab_t)
    return out.reshape(TGRID * TBLK, DIM)


def _permute_tokens(tok):
    g = tok // TBLK
    w = tok % TBLK
    return (g * TROWS + w % TROWS) * 4 + w // TROWS


def kernel(tokens_idx, emb_weight):
    tok = _permute_tokens(tokens_idx.reshape(-1).astype(jnp.int32))
    tab = _linearize_table(emb_weight)
    mesh = plsc.VectorSubcoreMesh(core_axis_name="c", subcore_axis_name="s")
    f = pl.kernel(
        _body,
        out_type=jax.ShapeDtypeStruct((B, DIM), jnp.float32),
        mesh=mesh,
        compiler_params=pltpu.CompilerParams(use_tc_tiling_on_sc=False),
        scratch_types=[
            pltpu.VMEM((RPC,), jnp.int32),
            pltpu.VMEM((RPC,), jnp.int32),
            pltpu.VMEM((RPC, DIM), jnp.float32),
            pltpu.VMEM((RPC, DIM), jnp.float32),
            pltpu.VMEM((CB, DIM), jnp.float32),
            pltpu.VMEM((CB, DIM), jnp.float32),
            pltpu.SemaphoreType.DMA,
            pltpu.SemaphoreType.DMA,
        ],
    )
    return f(tok, tab)


# speedup vs baseline: 3.0558x; 1.0754x over previous
import jax
import jax.numpy as jnp
from jax import lax
from jax.experimental import pallas as pl
from jax.experimental.pallas import tpu as pltpu
from jax.experimental.pallas import tpu_sc as plsc

NC, NS = 2, 16
NW = NC * NS
B, L, DIM = 16384, 50, 32
NUM_ROWS = 1000000
BAGS_W = B // NW
CB = 32
NCH = BAGS_W // CB
NST = NCH // 2
RPC = CB * L
INV_L = 1.0 / L


def _body(tok_hbm, tab_hbm, out_hbm,
          idx0, idx1, rows0, rows1, out0, out1, sem0, sem1):
    wid = lax.axis_index("s") * NC + lax.axis_index("c")
    idx = (idx0, idx1)
    rows = (rows0, rows1)
    outs = (out0, out1)
    sems = (sem0, sem1)

    def fire(c, p):
        bag0 = wid * BAGS_W + c * CB
        pltpu.sync_copy(tok_hbm.at[pl.ds(bag0 * L, RPC)], idx[p])
        pltpu.async_copy(tab_hbm.at[idx[p]], rows[p], sems[p])

    def process(c, p):
        pltpu.make_async_copy(tab_hbm.at[idx[p]], rows[p], sems[p]).wait()
        rv = rows[p]
        ov = outs[p]

        def bag(b, carry):
            a0 = rv[b * L, 0:16]
            a1 = rv[b * L, 16:32]
            for r in range(1, L):
                a0 = a0 + rv[b * L + r, 0:16]
                a1 = a1 + rv[b * L + r, 16:32]
            ov[b, 0:16] = a0 * INV_L
            ov[b, 16:32] = a1 * INV_L
            return carry

        lax.fori_loop(0, CB, bag, 0)
        bag0 = wid * BAGS_W + c * CB
        pltpu.sync_copy(ov, out_hbm.at[pl.ds(bag0, CB)])

    fire(0, 0)

    def step(s, carry):
        c0 = s * 2
        fire(c0 + 1, 1)
        process(c0, 0)

        @pl.when(s < NST - 1)
        def _():
            fire(c0 + 2, 0)

        process(c0 + 1, 1)
        return carry

    lax.fori_loop(0, NST, step, 0)


TBLK = 32768
TGRID = -(-NUM_ROWS // TBLK)
TROWS = TBLK * DIM // 128


def _transpose_body(x_ref, o_ref):
    x = x_ref[...]
    xx = jnp.concatenate(
        [x[:, TROWS * j:TROWS * (j + 1)] for j in range(4)], axis=0)
    o_ref[...] = jnp.transpose(xx)


def _linearize_table(emb_weight):
    tab_t = emb_weight.T
    out = pl.pallas_call(
        _transpose_body,
        grid=(TGRID,),
        in_specs=[pl.BlockSpec((DIM, TBLK), lambda g: (0, g))],
        out_specs=pl.BlockSpec((TROWS, 128), lambda g: (g, 0)),
        out_shape=jax.ShapeDtypeStruct((TGRID * TROWS, 128), jnp.float32),
    )(tab_t)
    return out.reshape(TGRID * TBLK, DIM)


def _permute_tokens(tok):
    g = tok // TBLK
    w = tok % TBLK
    return (g * TROWS + w % TROWS) * 4 + w // TROWS


def kernel(tokens_idx, emb_weight):
    tok = _permute_tokens(tokens_idx.reshape(-1).astype(jnp.int32))
    tab = _linearize_table(emb_weight)
    mesh = plsc.VectorSubcoreMesh(core_axis_name="c", subcore_axis_name="s")
    f = pl.kernel(
        _body,
        out_type=jax.ShapeDtypeStruct((B, DIM), jnp.float32),
        mesh=mesh,
        compiler_params=pltpu.CompilerParams(use_tc_tiling_on_sc=False),
        scratch_types=[
            pltpu.VMEM((RPC,), jnp.int32),
            pltpu.VMEM((RPC,), jnp.int32),
            pltpu.VMEM((RPC, DIM), jnp.float32),
            pltpu.VMEM((RPC, DIM), jnp.float32),
            pltpu.VMEM((CB, DIM), jnp.float32),
            pltpu.VMEM((CB, DIM), jnp.float32),
            pltpu.SemaphoreType.DMA,
            pltpu.SemaphoreType.DMA,
        ],
    )
    return f(tok, tab)


# speedup vs baseline: 3.0877x; 1.0104x over previous
import jax
import jax.numpy as jnp
from jax import lax
from jax.experimental import pallas as pl
from jax.experimental.pallas import tpu as pltpu
from jax.experimental.pallas import tpu_sc as plsc

NC, NS = 2, 16
NW = NC * NS
B, L, DIM = 16384, 50, 32
NUM_ROWS = 1000000
BAGS_W = B // NW
CB = 32
NCH = BAGS_W // CB
NST = NCH // 2
RPC = CB * L
INV_L = 1.0 / L


def _body(tok_hbm, tab_hbm, out_hbm,
          idx0, idx1, rows0, rows1, out0, out1, sem0, sem1):
    wid = lax.axis_index("s") * NC + lax.axis_index("c")
    idx = (idx0, idx1)
    rows = (rows0, rows1)
    outs = (out0, out1)
    sems = (sem0, sem1)

    def fire(c, p):
        bag0 = wid * BAGS_W + c * CB
        pltpu.sync_copy(tok_hbm.at[pl.ds(bag0 * L, RPC)], idx[p])
        pltpu.async_copy(tab_hbm.at[idx[p]], rows[p], sems[p])

    def process(c, p):
        pltpu.make_async_copy(tab_hbm.at[idx[p]], rows[p], sems[p]).wait()
        rv = rows[p]
        ov = outs[p]

        def bag(b, carry):
            a0 = rv[b * L, 0:16]
            a1 = rv[b * L, 16:32]
            for r in range(1, L):
                a0 = a0 + rv[b * L + r, 0:16]
                a1 = a1 + rv[b * L + r, 16:32]
            ov[b, 0:16] = a0 * INV_L
            ov[b, 16:32] = a1 * INV_L
            return carry

        lax.fori_loop(0, CB, bag, 0)
        bag0 = wid * BAGS_W + c * CB
        pltpu.sync_copy(ov, out_hbm.at[pl.ds(bag0, CB)])

    fire(0, 0)

    def step(s, carry):
        c0 = s * 2
        fire(c0 + 1, 1)
        process(c0, 0)

        @pl.when(s < NST - 1)
        def _():
            fire(c0 + 2, 0)

        process(c0 + 1, 1)
        return carry

    lax.fori_loop(0, NST, step, 0)


TBLK = 65536
TGRID = -(-NUM_ROWS // TBLK)
TROWS = TBLK * DIM // 128


def _transpose_body(x_ref, o_ref):
    x = x_ref[...]
    xx = jnp.concatenate(
        [x[:, TROWS * j:TROWS * (j + 1)] for j in range(4)], axis=0)
    o_ref[...] = jnp.transpose(xx)


def _linearize_table(emb_weight):
    tab_t = emb_weight.T
    out = pl.pallas_call(
        _transpose_body,
        grid=(TGRID,),
        in_specs=[pl.BlockSpec((DIM, TBLK), lambda g: (0, g))],
        out_specs=pl.BlockSpec((TROWS, 128), lambda g: (g, 0)),
        out_shape=jax.ShapeDtypeStruct((TGRID * TROWS, 128), jnp.float32),
    )(tab_t)
    return out.reshape(TGRID * TBLK, DIM)


def _permute_tokens(tok):
    g = tok // TBLK
    w = tok % TBLK
    return (g * TROWS + w % TROWS) * 4 + w // TROWS


def kernel(tokens_idx, emb_weight):
    tok = _permute_tokens(tokens_idx.reshape(-1).astype(jnp.int32))
    tab = _linearize_table(emb_weight)
    mesh = plsc.VectorSubcoreMesh(core_axis_name="c", subcore_axis_name="s")
    f = pl.kernel(
        _body,
        out_type=jax.ShapeDtypeStruct((B, DIM), jnp.float32),
        mesh=mesh,
        compiler_params=pltpu.CompilerParams(use_tc_tiling_on_sc=False),
        scratch_types=[
            pltpu.VMEM((RPC,), jnp.int32),
            pltpu.VMEM((RPC,), jnp.int32),
            pltpu.VMEM((RPC, DIM), jnp.float32),
            pltpu.VMEM((RPC, DIM), jnp.float32),
            pltpu.VMEM((CB, DIM), jnp.float32),
            pltpu.VMEM((CB, DIM), jnp.float32),
            pltpu.SemaphoreType.DMA,
            pltpu.SemaphoreType.DMA,
        ],
    )
    return f(tok, tab)
